# baseline (speedup 1.0000x reference)
import functools

import jax
import jax.numpy as jnp
from jax import lax
from jax.experimental import pallas as pl
from jax.experimental.pallas import tpu as pltpu
from jax.experimental.pallas import tpu_sc as plsc

N = 10000
E = 320000
D_ATOM = 64
D_FT = 64
D_IN = 128
H = 128
P_DIM = 64
NG = 256
NFT = 1000

NC = 2
NS = 16
NW = NC * NS

EC = 128
EPW_CHUNKS = 80
GRP = 16
E_PAD = NW * EPW_CHUNKS * EC

RPW = 320
N_PAD = NW * RPW
ROWS_PER_SUB = N_PAD // NS

_HI = jax.lax.Precision.DEFAULT

def _vector_mesh():
    return plsc.VectorSubcoreMesh(core_axis_name="c", subcore_axis_name="s")


@jax.jit
def _sc_ft_gather(ft, ftidx):
    @functools.partial(
        pl.kernel,
        out_type=jax.ShapeDtypeStruct((N_PAD, D_IN), jnp.float32),
        mesh=_vector_mesh(),
        scratch_types=[
            pltpu.VMEM((RPW,), jnp.int32),
            pltpu.VMEM((RPW, D_IN), jnp.float32),
        ],
    )
    def k(ft_hbm, idx_hbm, out_hbm, idx_v, rows_v):
        cid = lax.axis_index("c")
        sid = lax.axis_index("s")
        wid = sid * NC + cid
        base = wid * RPW
        pltpu.sync_copy(idx_hbm.at[pl.ds(base, RPW)], idx_v)
        for off, sz in ((0, 128), (128, 128), (256, 64)):
            pltpu.sync_copy(ft_hbm.at[idx_v.at[pl.ds(off, sz)]],
                            rows_v.at[pl.ds(off, sz)])
        pltpu.sync_copy(rows_v, out_hbm.at[pl.ds(base, RPW)])

    return k(ft, ftidx)


@jax.jit
def _sc_edge_agg(table, src, dst, zeros):
    d = table.shape[1]

    @functools.partial(
        pl.kernel,
        out_type=jax.ShapeDtypeStruct((NC, N_PAD, d), jnp.float32),
        mesh=_vector_mesh(),
        scratch_types=[
            pltpu.VMEM((GRP, EC), jnp.int32),
            pltpu.VMEM((GRP, EC), jnp.int32),
            pltpu.VMEM((EC, d), jnp.float32),
            pltpu.VMEM((EC, d), jnp.float32),
            pltpu.VMEM_SHARED((N_PAD, d), jnp.float32),
            pltpu.SemaphoreType.DMA,
            pltpu.SemaphoreType.DMA,
            pltpu.SemaphoreType.DMA,
            pltpu.SemaphoreType.DMA,
        ],
    )
    def k(table_hbm, src_hbm, dst_hbm, zeros_hbm, out_hbm,
          src_v, dst_v, rows0_v, rows1_v, agg_sh, sem0, sem1, ssem0, ssem1):
        cid = lax.axis_index("c")
        sid = lax.axis_index("s")
        wid = sid * NC + cid
        rbase = sid * ROWS_PER_SUB
        pltpu.sync_copy(zeros_hbm.at[pl.ds(rbase, ROWS_PER_SUB)],
                        agg_sh.at[pl.ds(rbase, ROWS_PER_SUB)])
        plsc.subcore_barrier()

        def gat(j, buf, sem):
            return pltpu.make_async_copy(table_hbm.at[src_v.at[j]], buf, sem)

        def sca(j, buf, sem):
            return pltpu.make_async_copy(buf, agg_sh.at[dst_v.at[j]], sem)

        @pl.loop(0, EPW_CHUNKS // GRP)
        def _(g):
            pltpu.sync_copy(src_hbm.at[wid, pl.ds(g * GRP, GRP)], src_v)
            pltpu.sync_copy(dst_hbm.at[wid, pl.ds(g * GRP, GRP)], dst_v)
            gat(0, rows0_v, sem0).start()
            gat(1, rows1_v, sem1).start()

            @pl.loop(0, GRP, step=2)
            def _(j):
                gat(j, rows0_v, sem0).wait()
                pltpu.async_copy(rows0_v, agg_sh.at[dst_v.at[j]], ssem0,
                                 add=True)
                gat(j + 1, rows1_v, sem1).wait()
                pltpu.async_copy(rows1_v, agg_sh.at[dst_v.at[j + 1]], ssem1,
                                 add=True)

                @pl.when(j + 2 < GRP)
                def _():
                    sca(j, rows0_v, ssem0).wait()
                    gat(j + 2, rows0_v, sem0).start()
                    sca(j + 1, rows1_v, ssem1).wait()
                    gat(j + 3, rows1_v, sem1).start()

                @pl.when(j + 2 >= GRP)
                def _():
                    sca(j, rows0_v, ssem0).wait()
                    sca(j + 1, rows1_v, ssem1).wait()

        plsc.subcore_barrier()
        pltpu.sync_copy(agg_sh.at[pl.ds(rbase, ROWS_PER_SUB)],
                        out_hbm.at[cid, pl.ds(rbase, ROWS_PER_SUB)])

    return k(table, src, dst, zeros)


def _dot_t(a, b_ref):
    return lax.dot_general(a, b_ref, (((1,), (1,)), ((), ())),
                           preferred_element_type=jnp.float32, precision=_HI)


def _concat_self_tc(x_pad, xf, w_self, b):
    blk = 1024

    def body(x_ref, f_ref, w_ref, b_ref, h_ref, s_ref):
        h0 = jnp.concatenate([x_ref[...], f_ref[:, :D_FT]], axis=1)
        h_ref[...] = h0
        s_ref[...] = _dot_t(h0, w_ref[...]) + b_ref[...]

    return pl.pallas_call(
        body,
        grid=(N_PAD // blk,),
        in_specs=[
            pl.BlockSpec((blk, D_ATOM), lambda i: (i, 0)),
            pl.BlockSpec((blk, D_IN), lambda i: (i, 0)),
            pl.BlockSpec((H, D_IN), lambda i: (0, 0)),
            pl.BlockSpec((1, H), lambda i: (0, 0)),
        ],
        out_specs=[
            pl.BlockSpec((blk, D_IN), lambda i: (i, 0)),
            pl.BlockSpec((blk, H), lambda i: (i, 0)),
        ],
        out_shape=[
            jax.ShapeDtypeStruct((N_PAD, D_IN), jnp.float32),
            jax.ShapeDtypeStruct((N_PAD, H), jnp.float32),
        ],
    )(x_pad, xf, w_self, b)


def _combine_self_tc(partials, s, w_l, w_self_next, b_next):
    blk = 1024

    def body(p_ref, s_ref, wl_ref, wn_ref, bn_ref, h_ref, sn_ref):
        agg = p_ref[0] + p_ref[1]
        t = _dot_t(agg, wl_ref[...]) + s_ref[...]
        h = jnp.where(t > 0, t, 0.1 * t)
        h_ref[...] = h
        sn_ref[...] = _dot_t(h, wn_ref[...]) + bn_ref[...]

    return pl.pallas_call(
        body,
        grid=(N_PAD // blk,),
        in_specs=[
            pl.BlockSpec((NC, blk, H), lambda i: (0, i, 0)),
            pl.BlockSpec((blk, H), lambda i: (i, 0)),
            pl.BlockSpec((H, H), lambda i: (0, 0)),
            pl.BlockSpec((H, H), lambda i: (0, 0)),
            pl.BlockSpec((1, H), lambda i: (0, 0)),
        ],
        out_specs=[
            pl.BlockSpec((blk, H), lambda i: (i, 0)),
            pl.BlockSpec((blk, H), lambda i: (i, 0)),
        ],
        out_shape=[
            jax.ShapeDtypeStruct((N_PAD, H), jnp.float32),
            jax.ShapeDtypeStruct((N_PAD, H), jnp.float32),
        ],
    )(partials, s, w_l, w_self_next, b_next)


def _combine_pool_head_tc(partials, s, w_l, batch_blocks, wp,
                          l1w, l1b, l2w, l2b):
    blk = 256
    nblk = N_PAD // blk

    def body(p_ref, s_ref, wl_ref, b_ref, wp_ref, l1w_ref, l1b_ref,
             l2w_ref, l2b_ref, o_ref, g_acc):
        i = pl.program_id(0)

        @pl.when(i == 0)
        def _():
            g_acc[...] = jnp.zeros_like(g_acc)

        agg = p_ref[0] + p_ref[1]
        t = _dot_t(agg, wl_ref[...]) + s_ref[...]
        h2 = jnp.where(t > 0, t, 0.1 * t)
        hp = _dot_t(h2, wp_ref[...])
        seg = b_ref[0, 0, :]
        onehot_t = (lax.broadcasted_iota(jnp.int32, (NG, blk), 0)
                    == seg[None, :]).astype(jnp.float32)
        g_acc[...] += lax.dot_general(
            onehot_t, hp, (((1,), (0,)), ((), ())),
            preferred_element_type=jnp.float32,
            precision=jax.lax.Precision.HIGHEST)

        @pl.when(i == nblk - 1)
        def _():
            g = g_acc[...]
            g = jnp.where(g > 0, g, 0.1 * g)
            a = _dot_t(g, l1w_ref[...]) + l1b_ref[...]
            a = jnp.maximum(a, 0.0)
            o = jnp.sum(a * l2w_ref[...], axis=1, keepdims=True)
            o_ref[...] = o + l2b_ref[...]

    return pl.pallas_call(
        body,
        grid=(nblk,),
        in_specs=[
            pl.BlockSpec((NC, blk, H), lambda i: (0, i, 0)),
            pl.BlockSpec((blk, H), lambda i: (i, 0)),
            pl.BlockSpec((H, H), lambda i: (0, 0)),
            pl.BlockSpec((1, 1, blk), lambda i: (i, 0, 0)),
            pl.BlockSpec((P_DIM, H), lambda i: (0, 0)),
            pl.BlockSpec((64, P_DIM), lambda i: (0, 0)),
            pl.BlockSpec((1, 64), lambda i: (0, 0)),
            pl.BlockSpec((1, 64), lambda i: (0, 0)),
            pl.BlockSpec((1, 1), lambda i: (0, 0)),
        ],
        out_specs=pl.BlockSpec((NG, 1), lambda i: (0, 0)),
        out_shape=jax.ShapeDtypeStruct((NG, 1), jnp.float32),
        scratch_shapes=[pltpu.VMEM((NG, P_DIM), jnp.float32)],
    )(partials, s, w_l, batch_blocks, wp, l1w, l1b, l2w, l2b)


def kernel(x, x_FT_index, edge_index, batch, FT_output,
           W1_l, W1_self, b1, W2_l, W2_self, b2,
           Wp, L1_W, L1_b, L2_W, L2_b):
    src = edge_index[0].astype(jnp.int32)
    dst = edge_index[1].astype(jnp.int32)
    pad_e = E_PAD - E
    pad_ids = jnp.arange(pad_e, dtype=jnp.int32)
    src_p = jnp.concatenate([src, pad_ids % N]).reshape(NW, EPW_CHUNKS, EC)
    dst_p = jnp.concatenate([dst, N + pad_ids % (N_PAD - N)]
                            ).reshape(NW, EPW_CHUNKS, EC)
    ftidx_p = jnp.pad(x_FT_index.astype(jnp.int32), (0, N_PAD - N))
    x_pad = jnp.pad(x, ((0, N_PAD - N), (0, 0)))
    batch_p = jnp.concatenate(
        [batch.astype(jnp.int32),
         jnp.full((N_PAD - N,), NG, jnp.int32)]).reshape(N_PAD // 256, 1, 256)
    zeros = jnp.zeros((N_PAD, D_IN), jnp.float32)
    b1r = b1.reshape(1, H)
    b2r = b2.reshape(1, H)
    l1b = L1_b.reshape(1, 64)
    l2b = L2_b.reshape(1, 1)

    ft128 = jnp.pad(FT_output, ((0, 0), (0, D_IN - D_FT)))
    xf = _sc_ft_gather(ft128, ftidx_p)
    h0, s1 = _concat_self_tc(x_pad, xf, W1_self, b1r)

    parts1 = _sc_edge_agg(h0, src_p, dst_p, zeros)
    h1, s2 = _combine_self_tc(parts1, s1, W1_l, W2_self, b2r)

    parts2 = _sc_edge_agg(h1, src_p, dst_p, zeros)
    return _combine_pool_head_tc(parts2, s2, W2_l, batch_p, Wp,
                                 L1_W, l1b, L2_W, l2b)


# speedup vs baseline: 1.2353x; 1.2353x over previous
import functools

import jax
import jax.numpy as jnp
from jax import lax
from jax.experimental import pallas as pl
from jax.experimental.pallas import tpu as pltpu
from jax.experimental.pallas import tpu_sc as plsc

N = 10000
E = 320000
D_ATOM = 64
D_FT = 64
D_IN = 128
H = 128
P_DIM = 64
NG = 256
NFT = 1000

NC = 2
NS = 16
NW = NC * NS

EC = 128
EPW_CHUNKS = 80
GRP = 40
E_PAD = NW * EPW_CHUNKS * EC

RPW = 320
N_PAD = NW * RPW
ROWS_PER_SUB = N_PAD // NS

_HI = jax.lax.Precision.DEFAULT

def _vector_mesh():
    return plsc.VectorSubcoreMesh(core_axis_name="c", subcore_axis_name="s")


@jax.jit
def _sc_ft_gather(ft, ftidx):
    @functools.partial(
        pl.kernel,
        out_type=jax.ShapeDtypeStruct((N_PAD, D_IN), jnp.float32),
        mesh=_vector_mesh(),
        scratch_types=[
            pltpu.VMEM((RPW,), jnp.int32),
            pltpu.VMEM((RPW, D_IN), jnp.float32),
            pltpu.SemaphoreType.DMA,
        ],
    )
    def k(ft_hbm, idx_hbm, out_hbm, idx_v, rows_v, sem):
        cid = lax.axis_index("c")
        sid = lax.axis_index("s")
        wid = sid * NC + cid
        base = wid * RPW
        pltpu.sync_copy(idx_hbm.at[pl.ds(base, RPW)], idx_v)
        cps = [
            pltpu.make_async_copy(ft_hbm.at[idx_v.at[pl.ds(off, sz)]],
                                  rows_v.at[pl.ds(off, sz)], sem)
            for off, sz in ((0, 128), (128, 128), (256, 64))
        ]
        for cp in cps:
            cp.start()
        for cp in cps:
            cp.wait()
        pltpu.sync_copy(rows_v, out_hbm.at[pl.ds(base, RPW)])

    return k(ft, ftidx)


@jax.jit
def _sc_edge_agg(table, src, dst, zeros):
    d = table.shape[1]

    @functools.partial(
        pl.kernel,
        out_type=jax.ShapeDtypeStruct((NC, N_PAD, d), jnp.float32),
        mesh=_vector_mesh(),
        scratch_types=[
            pltpu.VMEM((GRP, EC), jnp.int32),
            pltpu.VMEM((GRP, EC), jnp.int32),
            pltpu.VMEM((EC, d), jnp.float32),
            pltpu.VMEM((EC, d), jnp.float32),
            pltpu.VMEM_SHARED((N_PAD, d), jnp.float32),
            pltpu.SemaphoreType.DMA,
            pltpu.SemaphoreType.DMA,
        ],
    )
    def k(table_hbm, src_hbm, dst_hbm, zeros_hbm, out_hbm,
          src_v, dst_v, rows0_v, rows1_v, agg_sh, sem0, sem1):
        cid = lax.axis_index("c")
        sid = lax.axis_index("s")
        wid = sid * NC + cid
        rbase = sid * ROWS_PER_SUB
        pltpu.sync_copy(zeros_hbm.at[pl.ds(rbase, ROWS_PER_SUB)],
                        agg_sh.at[pl.ds(rbase, ROWS_PER_SUB)])
        plsc.subcore_barrier()

        def gat(j, buf, sem):
            return pltpu.make_async_copy(table_hbm.at[src_v.at[j]], buf, sem)

        @pl.loop(0, EPW_CHUNKS // GRP)
        def _(g):
            pltpu.sync_copy(src_hbm.at[wid, pl.ds(g * GRP, GRP)], src_v)
            pltpu.sync_copy(dst_hbm.at[wid, pl.ds(g * GRP, GRP)], dst_v)
            gat(0, rows0_v, sem0).start()

            @pl.loop(0, GRP, step=2)
            def _(j):
                gat(j + 1, rows1_v, sem1).start()
                gat(j, rows0_v, sem0).wait()
                pltpu.sync_copy(rows0_v, agg_sh.at[dst_v.at[j]], add=True)

                @pl.when(j + 2 < GRP)
                def _():
                    gat(j + 2, rows0_v, sem0).start()

                gat(j + 1, rows1_v, sem1).wait()
                pltpu.sync_copy(rows1_v, agg_sh.at[dst_v.at[j + 1]], add=True)

        plsc.subcore_barrier()
        pltpu.sync_copy(agg_sh.at[pl.ds(rbase, ROWS_PER_SUB)],
                        out_hbm.at[cid, pl.ds(rbase, ROWS_PER_SUB)])

    return k(table, src, dst, zeros)


def _dot_t(a, b_ref):
    return lax.dot_general(a, b_ref, (((1,), (1,)), ((), ())),
                           preferred_element_type=jnp.float32, precision=_HI)


def _concat_self_tc(x_pad, xf, w_self, b):
    blk = 1024

    def body(x_ref, f_ref, w_ref, b_ref, h_ref, s_ref):
        h0 = jnp.concatenate([x_ref[...], f_ref[:, :D_FT]], axis=1)
        h_ref[...] = h0
        s_ref[...] = _dot_t(h0, w_ref[...]) + b_ref[...]

    return pl.pallas_call(
        body,
        grid=(N_PAD // blk,),
        in_specs=[
            pl.BlockSpec((blk, D_ATOM), lambda i: (i, 0)),
            pl.BlockSpec((blk, D_IN), lambda i: (i, 0)),
            pl.BlockSpec((H, D_IN), lambda i: (0, 0)),
            pl.BlockSpec((1, H), lambda i: (0, 0)),
        ],
        out_specs=[
            pl.BlockSpec((blk, D_IN), lambda i: (i, 0)),
            pl.BlockSpec((blk, H), lambda i: (i, 0)),
        ],
        out_shape=[
            jax.ShapeDtypeStruct((N_PAD, D_IN), jnp.float32),
            jax.ShapeDtypeStruct((N_PAD, H), jnp.float32),
        ],
    )(x_pad, xf, w_self, b)


def _combine_self_tc(partials, s, w_l, w_self_next, b_next):
    blk = 1024

    def body(p_ref, s_ref, wl_ref, wn_ref, bn_ref, h_ref, sn_ref):
        agg = p_ref[0] + p_ref[1]
        t = _dot_t(agg, wl_ref[...]) + s_ref[...]
        h = jnp.where(t > 0, t, 0.1 * t)
        h_ref[...] = h
        sn_ref[...] = _dot_t(h, wn_ref[...]) + bn_ref[...]

    return pl.pallas_call(
        body,
        grid=(N_PAD // blk,),
        in_specs=[
            pl.BlockSpec((NC, blk, H), lambda i: (0, i, 0)),
            pl.BlockSpec((blk, H), lambda i: (i, 0)),
            pl.BlockSpec((H, H), lambda i: (0, 0)),
            pl.BlockSpec((H, H), lambda i: (0, 0)),
            pl.BlockSpec((1, H), lambda i: (0, 0)),
        ],
        out_specs=[
            pl.BlockSpec((blk, H), lambda i: (i, 0)),
            pl.BlockSpec((blk, H), lambda i: (i, 0)),
        ],
        out_shape=[
            jax.ShapeDtypeStruct((N_PAD, H), jnp.float32),
            jax.ShapeDtypeStruct((N_PAD, H), jnp.float32),
        ],
    )(partials, s, w_l, w_self_next, b_next)


def _combine_pool_head_tc(partials, s, w_l, batch_blocks, wp,
                          l1w, l1b, l2w, l2b):
    blk = 256
    nblk = N_PAD // blk

    def body(p_ref, s_ref, wl_ref, b_ref, wp_ref, l1w_ref, l1b_ref,
             l2w_ref, l2b_ref, o_ref, g_acc):
        i = pl.program_id(0)

        @pl.when(i == 0)
        def _():
            g_acc[...] = jnp.zeros_like(g_acc)

        agg = p_ref[0] + p_ref[1]
        t = _dot_t(agg, wl_ref[...]) + s_ref[...]
        h2 = jnp.where(t > 0, t, 0.1 * t)
        hp = _dot_t(h2, wp_ref[...])
        seg = b_ref[0, 0, :]
        onehot_t = (lax.broadcasted_iota(jnp.int32, (NG, blk), 0)
                    == seg[None, :]).astype(jnp.float32)
        g_acc[...] += lax.dot_general(
            onehot_t, hp, (((1,), (0,)), ((), ())),
            preferred_element_type=jnp.float32,
            precision=jax.lax.Precision.HIGHEST)

        @pl.when(i == nblk - 1)
        def _():
            g = g_acc[...]
            g = jnp.where(g > 0, g, 0.1 * g)
            a = _dot_t(g, l1w_ref[...]) + l1b_ref[...]
            a = jnp.maximum(a, 0.0)
            o = jnp.sum(a * l2w_ref[...], axis=1, keepdims=True)
            o_ref[...] = o + l2b_ref[...]

    return pl.pallas_call(
        body,
        grid=(nblk,),
        in_specs=[
            pl.BlockSpec((NC, blk, H), lambda i: (0, i, 0)),
            pl.BlockSpec((blk, H), lambda i: (i, 0)),
            pl.BlockSpec((H, H), lambda i: (0, 0)),
            pl.BlockSpec((1, 1, blk), lambda i: (i, 0, 0)),
            pl.BlockSpec((P_DIM, H), lambda i: (0, 0)),
            pl.BlockSpec((64, P_DIM), lambda i: (0, 0)),
            pl.BlockSpec((1, 64), lambda i: (0, 0)),
            pl.BlockSpec((1, 64), lambda i: (0, 0)),
            pl.BlockSpec((1, 1), lambda i: (0, 0)),
        ],
        out_specs=pl.BlockSpec((NG, 1), lambda i: (0, 0)),
        out_shape=jax.ShapeDtypeStruct((NG, 1), jnp.float32),
        scratch_shapes=[pltpu.VMEM((NG, P_DIM), jnp.float32)],
    )(partials, s, w_l, batch_blocks, wp, l1w, l1b, l2w, l2b)


def kernel(x, x_FT_index, edge_index, batch, FT_output,
           W1_l, W1_self, b1, W2_l, W2_self, b2,
           Wp, L1_W, L1_b, L2_W, L2_b):
    src = edge_index[0].astype(jnp.int32)
    dst = edge_index[1].astype(jnp.int32)
    pad_e = E_PAD - E
    pad_ids = jnp.arange(pad_e, dtype=jnp.int32)
    src_p = jnp.concatenate([src, pad_ids % N]).reshape(NW, EPW_CHUNKS, EC)
    dst_p = jnp.concatenate([dst, N + pad_ids % (N_PAD - N)]
                            ).reshape(NW, EPW_CHUNKS, EC)
    ftidx_p = jnp.pad(x_FT_index.astype(jnp.int32), (0, N_PAD - N))
    x_pad = jnp.pad(x, ((0, N_PAD - N), (0, 0)))
    batch_p = jnp.concatenate(
        [batch.astype(jnp.int32),
         jnp.full((N_PAD - N,), NG, jnp.int32)]).reshape(N_PAD // 256, 1, 256)
    zeros = jnp.zeros((N_PAD, D_IN), jnp.float32)
    b1r = b1.reshape(1, H)
    b2r = b2.reshape(1, H)
    l1b = L1_b.reshape(1, 64)
    l2b = L2_b.reshape(1, 1)

    ft128 = jnp.pad(FT_output, ((0, 0), (0, D_IN - D_FT)))
    xf = _sc_ft_gather(ft128, ftidx_p)
    h0, s1 = _concat_self_tc(x_pad, xf, W1_self, b1r)

    parts1 = _sc_edge_agg(h0, src_p, dst_p, zeros)
    h1, s2 = _combine_self_tc(parts1, s1, W1_l, W2_self, b2r)

    parts2 = _sc_edge_agg(h1, src_p, dst_p, zeros)
    return _combine_pool_head_tc(parts2, s2, W2_l, batch_p, Wp,
                                 L1_W, l1b, L2_W, l2b)


# speedup vs baseline: 1.2789x; 1.0353x over previous
import functools

import jax
import jax.numpy as jnp
from jax import lax
from jax.experimental import pallas as pl
from jax.experimental.pallas import tpu as pltpu
from jax.experimental.pallas import tpu_sc as plsc

N = 10000
E = 320000
D_ATOM = 64
D_FT = 64
D_IN = 128
H = 128
P_DIM = 64
NG = 256
NFT = 1000

NC = 2
NS = 16
NW = NC * NS

EC = 128
EPW_CHUNKS = 80
GRP = 40
E_PAD = NW * EPW_CHUNKS * EC

RPW = 320
N_PAD = NW * RPW
ROWS_PER_SUB = N_PAD // NS

_HI = jax.lax.Precision.DEFAULT

def _vector_mesh():
    return plsc.VectorSubcoreMesh(core_axis_name="c", subcore_axis_name="s")


@jax.jit
def _sc_build_h0(ft, ftidx, x_pad):
    @functools.partial(
        pl.kernel,
        out_type=jax.ShapeDtypeStruct((N_PAD, D_IN), jnp.float32),
        mesh=_vector_mesh(),
        scratch_types=[
            pltpu.VMEM((RPW,), jnp.int32),
            pltpu.VMEM((RPW, D_IN), jnp.float32),
            pltpu.VMEM((RPW, D_ATOM), jnp.float32),
            pltpu.SemaphoreType.DMA,
            pltpu.SemaphoreType.DMA,
        ],
    )
    def k(ft_hbm, idx_hbm, x_hbm, out_hbm, idx_v, rows_v, xrows_v, sem, xsem):
        cid = lax.axis_index("c")
        sid = lax.axis_index("s")
        wid = sid * NC + cid
        base = wid * RPW
        xcp = pltpu.make_async_copy(x_hbm.at[pl.ds(base, RPW)], xrows_v, xsem)
        xcp.start()
        pltpu.sync_copy(idx_hbm.at[pl.ds(base, RPW)], idx_v)
        cps = [
            pltpu.make_async_copy(ft_hbm.at[idx_v.at[pl.ds(off, sz)]],
                                  rows_v.at[pl.ds(off, sz)], sem)
            for off, sz in ((0, 128), (128, 128), (256, 64))
        ]
        for cp in cps:
            cp.start()
        xcp.wait()
        for cp in cps:
            cp.wait()

        @pl.loop(0, RPW)
        def _(r):
            for c in range(0, D_ATOM, 16):
                rows_v[r, pl.ds(c, 16)] = xrows_v[r, pl.ds(c, 16)]

        pltpu.sync_copy(rows_v, out_hbm.at[pl.ds(base, RPW)])

    return k(ft, ftidx, x_pad)


@jax.jit
def _sc_edge_agg(table, src, dst, zeros):
    d = table.shape[1]

    @functools.partial(
        pl.kernel,
        out_type=jax.ShapeDtypeStruct((NC, N_PAD, d), jnp.float32),
        mesh=_vector_mesh(),
        scratch_types=[
            pltpu.VMEM((GRP, EC), jnp.int32),
            pltpu.VMEM((GRP, EC), jnp.int32),
            pltpu.VMEM((EC, d), jnp.float32),
            pltpu.VMEM((EC, d), jnp.float32),
            pltpu.VMEM_SHARED((N_PAD, d), jnp.float32),
            pltpu.SemaphoreType.DMA,
            pltpu.SemaphoreType.DMA,
        ],
    )
    def k(table_hbm, src_hbm, dst_hbm, zeros_hbm, out_hbm,
          src_v, dst_v, rows0_v, rows1_v, agg_sh, sem0, sem1):
        cid = lax.axis_index("c")
        sid = lax.axis_index("s")
        wid = sid * NC + cid
        rbase = sid * ROWS_PER_SUB
        pltpu.sync_copy(zeros_hbm.at[pl.ds(rbase, ROWS_PER_SUB)],
                        agg_sh.at[pl.ds(rbase, ROWS_PER_SUB)])
        plsc.subcore_barrier()

        def gat(j, buf, sem):
            return pltpu.make_async_copy(table_hbm.at[src_v.at[j]], buf, sem)

        @pl.loop(0, EPW_CHUNKS // GRP)
        def _(g):
            pltpu.sync_copy(src_hbm.at[wid, pl.ds(g * GRP, GRP)], src_v)
            pltpu.sync_copy(dst_hbm.at[wid, pl.ds(g * GRP, GRP)], dst_v)
            gat(0, rows0_v, sem0).start()

            @pl.loop(0, GRP, step=2)
            def _(j):
                gat(j + 1, rows1_v, sem1).start()
                gat(j, rows0_v, sem0).wait()
                pltpu.sync_copy(rows0_v, agg_sh.at[dst_v.at[j]], add=True)

                @pl.when(j + 2 < GRP)
                def _():
                    gat(j + 2, rows0_v, sem0).start()

                gat(j + 1, rows1_v, sem1).wait()
                pltpu.sync_copy(rows1_v, agg_sh.at[dst_v.at[j + 1]], add=True)

        plsc.subcore_barrier()
        pltpu.sync_copy(agg_sh.at[pl.ds(rbase, ROWS_PER_SUB)],
                        out_hbm.at[cid, pl.ds(rbase, ROWS_PER_SUB)])

    return k(table, src, dst, zeros)


def _dot_t(a, b_ref):
    return lax.dot_general(a, b_ref, (((1,), (1,)), ((), ())),
                           preferred_element_type=jnp.float32, precision=_HI)


def _self_tc(h, w_self, b):
    blk = 1024

    def body(h_ref, w_ref, b_ref, s_ref):
        s_ref[...] = _dot_t(h_ref[...], w_ref[...]) + b_ref[...]

    return pl.pallas_call(
        body,
        grid=(N_PAD // blk,),
        in_specs=[
            pl.BlockSpec((blk, D_IN), lambda i: (i, 0)),
            pl.BlockSpec((H, D_IN), lambda i: (0, 0)),
            pl.BlockSpec((1, H), lambda i: (0, 0)),
        ],
        out_specs=pl.BlockSpec((blk, H), lambda i: (i, 0)),
        out_shape=jax.ShapeDtypeStruct((N_PAD, H), jnp.float32),
    )(h, w_self, b)


def _combine_self_tc(partials, s, w_l, w_self_next, b_next):
    blk = 1024

    def body(p_ref, s_ref, wl_ref, wn_ref, bn_ref, h_ref, sn_ref):
        agg = p_ref[0] + p_ref[1]
        t = _dot_t(agg, wl_ref[...]) + s_ref[...]
        h = jnp.where(t > 0, t, 0.1 * t)
        h_ref[...] = h
        sn_ref[...] = _dot_t(h, wn_ref[...]) + bn_ref[...]

    return pl.pallas_call(
        body,
        grid=(N_PAD // blk,),
        in_specs=[
            pl.BlockSpec((NC, blk, H), lambda i: (0, i, 0)),
            pl.BlockSpec((blk, H), lambda i: (i, 0)),
            pl.BlockSpec((H, H), lambda i: (0, 0)),
            pl.BlockSpec((H, H), lambda i: (0, 0)),
            pl.BlockSpec((1, H), lambda i: (0, 0)),
        ],
        out_specs=[
            pl.BlockSpec((blk, H), lambda i: (i, 0)),
            pl.BlockSpec((blk, H), lambda i: (i, 0)),
        ],
        out_shape=[
            jax.ShapeDtypeStruct((N_PAD, H), jnp.float32),
            jax.ShapeDtypeStruct((N_PAD, H), jnp.float32),
        ],
    )(partials, s, w_l, w_self_next, b_next)


def _combine_pool_head_tc(partials, s, w_l, batch_blocks, wp,
                          l1w, l1b, l2w, l2b):
    blk = 256
    nblk = N_PAD // blk

    def body(p_ref, s_ref, wl_ref, b_ref, wp_ref, l1w_ref, l1b_ref,
             l2w_ref, l2b_ref, o_ref, g_acc):
        i = pl.program_id(0)

        @pl.when(i == 0)
        def _():
            g_acc[...] = jnp.zeros_like(g_acc)

        agg = p_ref[0] + p_ref[1]
        t = _dot_t(agg, wl_ref[...]) + s_ref[...]
        h2 = jnp.where(t > 0, t, 0.1 * t)
        hp = _dot_t(h2, wp_ref[...])
        seg = b_ref[0, 0, :]
        onehot_t = (lax.broadcasted_iota(jnp.int32, (NG, blk), 0)
                    == seg[None, :]).astype(jnp.float32)
        g_acc[...] += lax.dot_general(
            onehot_t, hp, (((1,), (0,)), ((), ())),
            preferred_element_type=jnp.float32,
            precision=jax.lax.Precision.HIGHEST)

        @pl.when(i == nblk - 1)
        def _():
            g = g_acc[...]
            g = jnp.where(g > 0, g, 0.1 * g)
            a = _dot_t(g, l1w_ref[...]) + l1b_ref[...]
            a = jnp.maximum(a, 0.0)
            o = jnp.sum(a * l2w_ref[...], axis=1, keepdims=True)
            o_ref[...] = o + l2b_ref[...]

    return pl.pallas_call(
        body,
        grid=(nblk,),
        in_specs=[
            pl.BlockSpec((NC, blk, H), lambda i: (0, i, 0)),
            pl.BlockSpec((blk, H), lambda i: (i, 0)),
            pl.BlockSpec((H, H), lambda i: (0, 0)),
            pl.BlockSpec((1, 1, blk), lambda i: (i, 0, 0)),
            pl.BlockSpec((P_DIM, H), lambda i: (0, 0)),
            pl.BlockSpec((64, P_DIM), lambda i: (0, 0)),
            pl.BlockSpec((1, 64), lambda i: (0, 0)),
            pl.BlockSpec((1, 64), lambda i: (0, 0)),
            pl.BlockSpec((1, 1), lambda i: (0, 0)),
        ],
        out_specs=pl.BlockSpec((NG, 1), lambda i: (0, 0)),
        out_shape=jax.ShapeDtypeStruct((NG, 1), jnp.float32),
        scratch_shapes=[pltpu.VMEM((NG, P_DIM), jnp.float32)],
    )(partials, s, w_l, batch_blocks, wp, l1w, l1b, l2w, l2b)


def kernel(x, x_FT_index, edge_index, batch, FT_output,
           W1_l, W1_self, b1, W2_l, W2_self, b2,
           Wp, L1_W, L1_b, L2_W, L2_b):
    src = edge_index[0].astype(jnp.int32)
    dst = edge_index[1].astype(jnp.int32)
    pad_e = E_PAD - E
    pad_ids = jnp.arange(pad_e, dtype=jnp.int32)
    src_p = jnp.concatenate([src, pad_ids % N]).reshape(NW, EPW_CHUNKS, EC)
    dst_p = jnp.concatenate([dst, N + pad_ids % (N_PAD - N)]
                            ).reshape(NW, EPW_CHUNKS, EC)
    ftidx_p = jnp.pad(x_FT_index.astype(jnp.int32), (0, N_PAD - N))
    x_pad = jnp.pad(x, ((0, N_PAD - N), (0, 0)))
    batch_p = jnp.concatenate(
        [batch.astype(jnp.int32),
         jnp.full((N_PAD - N,), NG, jnp.int32)]).reshape(N_PAD // 256, 1, 256)
    zeros = jnp.zeros((N_PAD, D_IN), jnp.float32)
    b1r = b1.reshape(1, H)
    b2r = b2.reshape(1, H)
    l1b = L1_b.reshape(1, 64)
    l2b = L2_b.reshape(1, 1)

    ft128 = jnp.pad(FT_output, ((0, 0), (D_IN - D_FT, 0)))
    h0 = _sc_build_h0(ft128, ftidx_p, x_pad)

    parts1 = _sc_edge_agg(h0, src_p, dst_p, zeros)
    s1 = _self_tc(h0, W1_self, b1r)
    h1, s2 = _combine_self_tc(parts1, s1, W1_l, W2_self, b2r)

    parts2 = _sc_edge_agg(h1, src_p, dst_p, zeros)
    return _combine_pool_head_tc(parts2, s2, W2_l, batch_p, Wp,
                                 L1_W, l1b, L2_W, l2b)


# speedup vs baseline: 1.2827x; 1.0030x over previous
import functools

import jax
import jax.numpy as jnp
from jax import lax
from jax.experimental import pallas as pl
from jax.experimental.pallas import tpu as pltpu
from jax.experimental.pallas import tpu_sc as plsc

N = 10000
E = 320000
D_ATOM = 64
D_FT = 64
D_IN = 128
H = 128
P_DIM = 64
NG = 256
NFT = 1000

NC = 2
NS = 16
NW = NC * NS

EC = 128
EPW_CHUNKS = 80
GRP = 40
E_PAD = NW * EPW_CHUNKS * EC

RPW = 320
N_PAD = NW * RPW
ROWS_PER_SUB = N_PAD // NS

_HI = jax.lax.Precision.DEFAULT

def _vector_mesh():
    return plsc.VectorSubcoreMesh(core_axis_name="c", subcore_axis_name="s")


@jax.jit
def _sc_build_h0(ft, ftidx, x_pad):
    @functools.partial(
        pl.kernel,
        out_type=jax.ShapeDtypeStruct((N_PAD, D_IN), jnp.float32),
        mesh=_vector_mesh(),
        scratch_types=[
            pltpu.VMEM((RPW,), jnp.int32),
            pltpu.VMEM((RPW, D_IN), jnp.float32),
            pltpu.VMEM((RPW, D_ATOM), jnp.float32),
            pltpu.SemaphoreType.DMA,
            pltpu.SemaphoreType.DMA,
        ],
    )
    def k(ft_hbm, idx_hbm, x_hbm, out_hbm, idx_v, rows_v, xrows_v, sem, xsem):
        cid = lax.axis_index("c")
        sid = lax.axis_index("s")
        wid = sid * NC + cid
        base = wid * RPW
        xcp = pltpu.make_async_copy(x_hbm.at[pl.ds(base, RPW)], xrows_v, xsem)
        xcp.start()
        pltpu.sync_copy(idx_hbm.at[pl.ds(base, RPW)], idx_v)
        cps = [
            pltpu.make_async_copy(ft_hbm.at[idx_v.at[pl.ds(off, sz)]],
                                  rows_v.at[pl.ds(off, sz)], sem)
            for off, sz in ((0, 128), (128, 128), (256, 64))
        ]
        for cp in cps:
            cp.start()
        xcp.wait()
        for cp in cps:
            cp.wait()

        @pl.loop(0, RPW)
        def _(r):
            for c in range(0, D_ATOM, 16):
                rows_v[r, pl.ds(c, 16)] = xrows_v[r, pl.ds(c, 16)]

        pltpu.sync_copy(rows_v, out_hbm.at[pl.ds(base, RPW)])

    return k(ft, ftidx, x_pad)


@jax.jit
def _sc_edge_agg(table, src, dst, zeros):
    d = table.shape[1]

    @functools.partial(
        pl.kernel,
        out_type=jax.ShapeDtypeStruct((NC, N_PAD, d), jnp.float32),
        mesh=_vector_mesh(),
        scratch_types=[
            pltpu.VMEM((GRP, EC), jnp.int32),
            pltpu.VMEM((GRP, EC), jnp.int32),
            pltpu.VMEM((EC, d), jnp.float32),
            pltpu.VMEM((EC, d), jnp.float32),
            pltpu.VMEM_SHARED((N_PAD, d), jnp.float32),
            pltpu.SemaphoreType.DMA,
            pltpu.SemaphoreType.DMA,
        ],
    )
    def k(table_hbm, src_hbm, dst_hbm, zeros_hbm, out_hbm,
          src_v, dst_v, rows0_v, rows1_v, agg_sh, sem0, sem1):
        cid = lax.axis_index("c")
        sid = lax.axis_index("s")
        wid = sid * NC + cid
        rbase = sid * ROWS_PER_SUB
        pltpu.sync_copy(zeros_hbm.at[pl.ds(rbase, ROWS_PER_SUB)],
                        agg_sh.at[pl.ds(rbase, ROWS_PER_SUB)])
        plsc.subcore_barrier()

        def gat(j, buf, sem):
            return pltpu.make_async_copy(table_hbm.at[src_v.at[j]], buf, sem)

        @pl.loop(0, EPW_CHUNKS // GRP)
        def _(g):
            pltpu.sync_copy(src_hbm.at[wid, pl.ds(g * GRP, GRP)], src_v)
            pltpu.sync_copy(dst_hbm.at[wid, pl.ds(g * GRP, GRP)], dst_v)
            gat(0, rows0_v, sem0).start()

            @pl.loop(0, GRP, step=2)
            def _(j):
                gat(j + 1, rows1_v, sem1).start()
                gat(j, rows0_v, sem0).wait()
                pltpu.sync_copy(rows0_v, agg_sh.at[dst_v.at[j]], add=True)

                @pl.when(j + 2 < GRP)
                def _():
                    gat(j + 2, rows0_v, sem0).start()

                gat(j + 1, rows1_v, sem1).wait()
                pltpu.sync_copy(rows1_v, agg_sh.at[dst_v.at[j + 1]], add=True)

        plsc.subcore_barrier()
        pltpu.sync_copy(agg_sh.at[pl.ds(rbase, ROWS_PER_SUB)],
                        out_hbm.at[cid, pl.ds(rbase, ROWS_PER_SUB)])

    return k(table, src, dst, zeros)


def _dot_t(a, b_ref):
    return lax.dot_general(a, b_ref, (((1,), (1,)), ((), ())),
                           preferred_element_type=jnp.float32, precision=_HI)


def _layer_tc(partials, h_prev, w_l, w_self, b):
    blk = 1024

    def body(p_ref, h_ref, wl_ref, ws_ref, b_ref, o_ref):
        agg = p_ref[0] + p_ref[1]
        t = (_dot_t(agg, wl_ref[...]) + _dot_t(h_ref[...], ws_ref[...])
             + b_ref[...])
        o_ref[...] = jnp.where(t > 0, t, 0.1 * t)

    return pl.pallas_call(
        body,
        grid=(N_PAD // blk,),
        in_specs=[
            pl.BlockSpec((NC, blk, H), lambda i: (0, i, 0)),
            pl.BlockSpec((blk, D_IN), lambda i: (i, 0)),
            pl.BlockSpec((H, H), lambda i: (0, 0)),
            pl.BlockSpec((H, D_IN), lambda i: (0, 0)),
            pl.BlockSpec((1, H), lambda i: (0, 0)),
        ],
        out_specs=pl.BlockSpec((blk, H), lambda i: (i, 0)),
        out_shape=jax.ShapeDtypeStruct((N_PAD, H), jnp.float32),
    )(partials, h_prev, w_l, w_self, b)


def _combine_pool_head_tc(partials, h_prev, w_l, w_self, b, batch_blocks, wp,
                          l1w, l1b, l2w, l2b):
    blk = 256
    nblk = N_PAD // blk

    def body(p_ref, h_ref, wl_ref, ws_ref, bb_ref, b_ref, wp_ref,
             l1w_ref, l1b_ref, l2w_ref, l2b_ref, o_ref, g_acc):
        i = pl.program_id(0)

        @pl.when(i == 0)
        def _():
            g_acc[...] = jnp.zeros_like(g_acc)

        agg = p_ref[0] + p_ref[1]
        t = (_dot_t(agg, wl_ref[...]) + _dot_t(h_ref[...], ws_ref[...])
             + bb_ref[...])
        h2 = jnp.where(t > 0, t, 0.1 * t)
        hp = _dot_t(h2, wp_ref[...])
        seg = b_ref[0, 0, :]
        onehot_t = (lax.broadcasted_iota(jnp.int32, (NG, blk), 0)
                    == seg[None, :]).astype(jnp.float32)
        g_acc[...] += lax.dot_general(
            onehot_t, hp, (((1,), (0,)), ((), ())),
            preferred_element_type=jnp.float32,
            precision=jax.lax.Precision.HIGHEST)

        @pl.when(i == nblk - 1)
        def _():
            g = g_acc[...]
            g = jnp.where(g > 0, g, 0.1 * g)
            a = _dot_t(g, l1w_ref[...]) + l1b_ref[...]
            a = jnp.maximum(a, 0.0)
            o = jnp.sum(a * l2w_ref[...], axis=1, keepdims=True)
            o_ref[...] = o + l2b_ref[...]

    return pl.pallas_call(
        body,
        grid=(nblk,),
        in_specs=[
            pl.BlockSpec((NC, blk, H), lambda i: (0, i, 0)),
            pl.BlockSpec((blk, H), lambda i: (i, 0)),
            pl.BlockSpec((H, H), lambda i: (0, 0)),
            pl.BlockSpec((H, H), lambda i: (0, 0)),
            pl.BlockSpec((1, H), lambda i: (0, 0)),
            pl.BlockSpec((1, 1, blk), lambda i: (i, 0, 0)),
            pl.BlockSpec((P_DIM, H), lambda i: (0, 0)),
            pl.BlockSpec((64, P_DIM), lambda i: (0, 0)),
            pl.BlockSpec((1, 64), lambda i: (0, 0)),
            pl.BlockSpec((1, 64), lambda i: (0, 0)),
            pl.BlockSpec((1, 1), lambda i: (0, 0)),
        ],
        out_specs=pl.BlockSpec((NG, 1), lambda i: (0, 0)),
        out_shape=jax.ShapeDtypeStruct((NG, 1), jnp.float32),
        scratch_shapes=[pltpu.VMEM((NG, P_DIM), jnp.float32)],
    )(partials, h_prev, w_l, w_self, b, batch_blocks, wp, l1w, l1b, l2w, l2b)


def kernel(x, x_FT_index, edge_index, batch, FT_output,
           W1_l, W1_self, b1, W2_l, W2_self, b2,
           Wp, L1_W, L1_b, L2_W, L2_b):
    src = edge_index[0].astype(jnp.int32)
    dst = edge_index[1].astype(jnp.int32)
    pad_e = E_PAD - E
    pad_ids = jnp.arange(pad_e, dtype=jnp.int32)
    src_p = jnp.concatenate([src, pad_ids % N]).reshape(NW, EPW_CHUNKS, EC)
    dst_p = jnp.concatenate([dst, N + pad_ids % (N_PAD - N)]
                            ).reshape(NW, EPW_CHUNKS, EC)
    ftidx_p = jnp.pad(x_FT_index.astype(jnp.int32), (0, N_PAD - N))
    x_pad = jnp.pad(x, ((0, N_PAD - N), (0, 0)))
    batch_p = jnp.concatenate(
        [batch.astype(jnp.int32),
         jnp.full((N_PAD - N,), NG, jnp.int32)]).reshape(N_PAD // 256, 1, 256)
    zeros = jnp.zeros((N_PAD, D_IN), jnp.float32)
    b1r = b1.reshape(1, H)
    b2r = b2.reshape(1, H)
    l1b = L1_b.reshape(1, 64)
    l2b = L2_b.reshape(1, 1)

    ft128 = jnp.pad(FT_output, ((0, 0), (D_IN - D_FT, 0)))
    h0 = _sc_build_h0(ft128, ftidx_p, x_pad)

    parts1 = _sc_edge_agg(h0, src_p, dst_p, zeros)
    h1 = _layer_tc(parts1, h0, W1_l, W1_self, b1r)

    parts2 = _sc_edge_agg(h1, src_p, dst_p, zeros)
    return _combine_pool_head_tc(parts2, h1, W2_l, W2_self, b2r, batch_p, Wp,
                                 L1_W, l1b, L2_W, l2b)
